# baseline (speedup 1.0000x reference)
import jax
import jax.numpy as jnp
from jax.experimental import pallas as pl
from jax.experimental.pallas import tpu as pltpu

_N_LEVELS = 4
_N_E = 1024
_E_DIM = 32
_B = 16384
_BLOCK = 1024
_HALF = 512
_BETA = 0.25


def _rvq_kernel(x_ref, cb_ref, xq_ref, res_ref, loss_ref, idx_ref, dist_ref):
    n_h = _BLOCK // _HALF
    r = [x_ref[pl.ds(h * _HALF, _HALF), :] for h in range(n_h)]
    xq = [jnp.zeros_like(r[h]) for h in range(n_h)]
    loss_acc = jnp.float32(0.0)
    col = jax.lax.broadcasted_iota(jnp.int32, (_HALF, _N_E), 1)
    for lvl in range(_N_LEVELS):
        emb = cb_ref[lvl]
        emb_hi = emb.astype(jnp.bfloat16)
        emb_lo = (emb - emb_hi.astype(jnp.float32)).astype(jnp.bfloat16)
        emb_cat = jnp.concatenate([emb_hi, emb_lo], axis=1)
        b_norm = jnp.sum(emb * emb, axis=1)[None, :]
        for h in range(n_h):
            rh = r[h]
            cross2 = jax.lax.dot_general(
                rh + rh, emb, (((1,), (1,)), ((), ())),
                preferred_element_type=jnp.float32)
            d = (jnp.sum(rh * rh, axis=1, keepdims=True)
                 + b_norm
                 - cross2)
            dist_ref[pl.ds(h * _HALF, _HALF), lvl, :] = d
            mind = jnp.min(d, axis=1, keepdims=True)
            idx = jnp.min(jnp.where(d == mind, col, _N_E), axis=1)
            idx_ref[lvl, pl.ds(h * _HALF, _HALF)] = idx
            onehot = (col == idx[:, None]).astype(jnp.bfloat16)
            qcat = jax.lax.dot_general(
                onehot, emb_cat, (((1,), (0,)), ((), ())),
                preferred_element_type=jnp.float32)
            q = qcat[:, :_E_DIM] + qcat[:, _E_DIM:]
            diff = rh - q
            loss_acc = loss_acc + jnp.sum(diff * diff)
            x_res = rh + (q - rh)
            xq[h] = xq[h] + x_res
            r[h] = rh - x_res
    for h in range(n_h):
        xq_ref[pl.ds(h * _HALF, _HALF), :] = xq[h]
        res_ref[pl.ds(h * _HALF, _HALF), :] = r[h]
    loss_ref[...] = jnp.full((1, 1, 1), loss_acc, jnp.float32)


def kernel(x, codebooks):
    grid = _B // _BLOCK
    out_shape = (
        jax.ShapeDtypeStruct((_B, _E_DIM), jnp.float32),
        jax.ShapeDtypeStruct((_B, _E_DIM), jnp.float32),
        jax.ShapeDtypeStruct((grid, 1, 1), jnp.float32),
        jax.ShapeDtypeStruct((_N_LEVELS, _B), jnp.int32),
        jax.ShapeDtypeStruct((_B, _N_LEVELS, _N_E), jnp.float32)
    )
    xq, res, loss_sum, idx_t, dist = pl.pallas_call(
        _rvq_kernel,
        grid=(grid,),
        in_specs=[
            pl.BlockSpec((_BLOCK, _E_DIM), lambda i: (i, 0)),
            pl.BlockSpec((_N_LEVELS, _N_E, _E_DIM), lambda i: (0, 0, 0)),
        ],
        out_specs=(
            pl.BlockSpec((_BLOCK, _E_DIM), lambda i: (i, 0)),
            pl.BlockSpec((_BLOCK, _E_DIM), lambda i: (i, 0)),
            pl.BlockSpec((1, 1, 1), lambda i: (i, 0, 0)),
            pl.BlockSpec((_N_LEVELS, _BLOCK), lambda i: (0, i)),
            pl.BlockSpec((_BLOCK, _N_LEVELS, _N_E), lambda i: (i, 0, 0)),
        ),
        out_shape=out_shape,
        compiler_params=pltpu.CompilerParams(
            dimension_semantics=("parallel",)),
    )(x, codebooks)
    mean_losses = jnp.sum(loss_sum) * (
        (1.0 + _BETA) / (_N_LEVELS * _B * _E_DIM))
    return (xq, res, mean_losses, idx_t.T, dist)


# speedup vs baseline: 1.0621x; 1.0621x over previous
import jax
import jax.numpy as jnp
from jax.experimental import pallas as pl
from jax.experimental.pallas import tpu as pltpu

_N_LEVELS = 4
_N_E = 1024
_E_DIM = 32
_B = 16384
_BLOCK = 1024
_HALF = 512
_BETA = 0.25


def _rvq_kernel(x_ref, cb_ref, xq_ref, res_ref, loss_ref, idx_ref, dist_ref):
    r = x_ref[...]
    xq = jnp.zeros_like(r)
    loss_acc = jnp.float32(0.0)
    col = jax.lax.broadcasted_iota(jnp.int32, (_BLOCK, _N_E), 1)
    for lvl in range(_N_LEVELS):
        emb = cb_ref[lvl]
        emb_hi = emb.astype(jnp.bfloat16)
        emb_lo = (emb - emb_hi.astype(jnp.float32)).astype(jnp.bfloat16)
        emb_cat = jnp.concatenate([emb_hi, emb_lo], axis=1)
        cross2 = jax.lax.dot_general(
            r + r, emb, (((1,), (1,)), ((), ())),
            preferred_element_type=jnp.float32)
        d = (jnp.sum(r * r, axis=1, keepdims=True)
             + jnp.sum(emb * emb, axis=1)[None, :]
             - cross2)
        dist_ref[:, lvl, :] = d
        mind = jnp.min(d, axis=1, keepdims=True)
        idx = jnp.min(jnp.where(d == mind, col, _N_E), axis=1)
        idx_ref[lvl, :] = idx
        onehot = (col == idx[:, None]).astype(jnp.bfloat16)
        qcat = jax.lax.dot_general(
            onehot, emb_cat, (((1,), (0,)), ((), ())),
            preferred_element_type=jnp.float32)
        q = qcat[:, :_E_DIM] + qcat[:, _E_DIM:]
        diff = r - q
        loss_acc = loss_acc + jnp.sum(diff * diff)
        x_res = r + (q - r)
        xq = xq + x_res
        r = r - x_res
    xq_ref[...] = xq
    res_ref[...] = r
    loss_ref[...] = jnp.full((1, 1, 1), loss_acc, jnp.float32)


def kernel(x, codebooks):
    grid = _B // _BLOCK
    out_shape = (
        jax.ShapeDtypeStruct((_B, _E_DIM), jnp.float32),
        jax.ShapeDtypeStruct((_B, _E_DIM), jnp.float32),
        jax.ShapeDtypeStruct((grid, 1, 1), jnp.float32),
        jax.ShapeDtypeStruct((_N_LEVELS, _B), jnp.int32),
        jax.ShapeDtypeStruct((_B, _N_LEVELS, _N_E), jnp.float32)
    )
    xq, res, loss_sum, idx_t, dist = pl.pallas_call(
        _rvq_kernel,
        grid=(grid,),
        in_specs=[
            pl.BlockSpec((_BLOCK, _E_DIM), lambda i: (i, 0)),
            pl.BlockSpec((_N_LEVELS, _N_E, _E_DIM), lambda i: (0, 0, 0)),
        ],
        out_specs=(
            pl.BlockSpec((_BLOCK, _E_DIM), lambda i: (i, 0)),
            pl.BlockSpec((_BLOCK, _E_DIM), lambda i: (i, 0)),
            pl.BlockSpec((1, 1, 1), lambda i: (i, 0, 0)),
            pl.BlockSpec((_N_LEVELS, _BLOCK), lambda i: (0, i)),
            pl.BlockSpec((_BLOCK, _N_LEVELS, _N_E), lambda i: (i, 0, 0)),
        ),
        out_shape=out_shape,
        compiler_params=pltpu.CompilerParams(
            dimension_semantics=("parallel",)),
    )(x, codebooks)
    mean_losses = jnp.sum(loss_sum) * (
        (1.0 + _BETA) / (_N_LEVELS * _B * _E_DIM))
    return (xq, res, mean_losses, idx_t.T, dist)


# speedup vs baseline: 1.0687x; 1.0062x over previous
import jax
import jax.numpy as jnp
from jax.experimental import pallas as pl

_N_LEVELS = 4
_N_E = 1024
_E_DIM = 32
_B = 16384
_BLOCK = 1024
_BETA = 0.25


def _rvq_kernel(x_ref, cb_ref, xq_ref, res_ref, loss_ref, idx_ref, dist_ref):
    r = x_ref[...]
    xq = jnp.zeros_like(r)
    loss_acc = jnp.float32(0.0)
    col = jax.lax.broadcasted_iota(jnp.int32, (_BLOCK, _N_E), 1)
    for lvl in range(_N_LEVELS):
        emb = cb_ref[lvl]
        emb_hi = emb.astype(jnp.bfloat16)
        emb_lo = (emb - emb_hi.astype(jnp.float32)).astype(jnp.bfloat16)
        emb_cat = jnp.concatenate([emb_hi, emb_lo], axis=1)
        cross2 = jax.lax.dot_general(
            r + r, emb, (((1,), (1,)), ((), ())),
            preferred_element_type=jnp.float32)
        d = (jnp.sum(r * r, axis=1, keepdims=True)
             + jnp.sum(emb * emb, axis=1)[None, :]
             - cross2)
        dist_ref[:, lvl, :] = d
        mind = jnp.min(d, axis=1, keepdims=True)
        idx = jnp.min(jnp.where(d == mind, col, _N_E), axis=1)
        idx_ref[lvl, :] = idx
        onehot = (col == idx[:, None]).astype(jnp.bfloat16)
        qcat = jax.lax.dot_general(
            onehot, emb_cat, (((1,), (0,)), ((), ())),
            preferred_element_type=jnp.float32)
        q = qcat[:, :_E_DIM] + qcat[:, _E_DIM:]
        diff = r - q
        loss_acc = loss_acc + jnp.sum(diff * diff)
        x_res = r + (q - r)
        xq = xq + x_res
        r = r - x_res
    xq_ref[...] = xq
    res_ref[...] = r

    @pl.when(pl.program_id(0) == 0)
    def _init():
        loss_ref[...] = jnp.zeros((1, 1), jnp.float32)

    loss_ref[...] = loss_ref[...] + loss_acc


def kernel(x, codebooks):
    grid = _B // _BLOCK
    out_shape = (
        jax.ShapeDtypeStruct((_B, _E_DIM), jnp.float32),
        jax.ShapeDtypeStruct((_B, _E_DIM), jnp.float32),
        jax.ShapeDtypeStruct((1, 1), jnp.float32),
        jax.ShapeDtypeStruct((_N_LEVELS, _B), jnp.int32),
        jax.ShapeDtypeStruct((_B, _N_LEVELS, _N_E), jnp.float32)
    )
    xq, res, loss_sum, idx_t, dist = pl.pallas_call(
        _rvq_kernel,
        grid=(grid,),
        in_specs=[
            pl.BlockSpec((_BLOCK, _E_DIM), lambda i: (i, 0)),
            pl.BlockSpec((_N_LEVELS, _N_E, _E_DIM), lambda i: (0, 0, 0)),
        ],
        out_specs=(
            pl.BlockSpec((_BLOCK, _E_DIM), lambda i: (i, 0)),
            pl.BlockSpec((_BLOCK, _E_DIM), lambda i: (i, 0)),
            pl.BlockSpec((1, 1), lambda i: (0, 0)),
            pl.BlockSpec((_N_LEVELS, _BLOCK), lambda i: (0, i)),
            pl.BlockSpec((_BLOCK, _N_LEVELS, _N_E), lambda i: (i, 0, 0)),
        ),
        out_shape=out_shape,
    )(x, codebooks)
    mean_losses = loss_sum[0, 0] * (
        (1.0 + _BETA) / (_N_LEVELS * _B * _E_DIM))
    return (xq, res, mean_losses, idx_t.T, dist)
